# Initial kernel scaffold; baseline (speedup 1.0000x reference)
import functools

import jax
import jax.numpy as jnp
from jax.experimental import pallas as pl
from jax.experimental.pallas import tpu as pltpu

CUTOFF = 9.0
SIGMA = 3.405
EPSILON = 0.238

N_BLOCK = 256


def _pressure_kernel(params_ref, qxr, qyr, qzr, qxc, qyc, qzc,
                     vx, vy, vz, m, out_ref, *, n_atoms, n_block):
    rb = pl.program_id(1)
    ldx = params_ref[0]
    ldy = params_ref[1]
    ldz = params_ref[2]
    inv_ldx = params_ref[3]
    inv_ldy = params_ref[4]
    inv_ldz = params_ref[5]
    kin_fac = params_ref[6]
    vir_fac = params_ref[7]

    rx = qxr[0]
    ry = qyr[0]
    rz = qzr[0]
    cx = qxc[0]
    cy = qyc[0]
    cz = qzc[0]

    def wrapped(c, r, ld, inv_ld):
        d = c - r
        off = jnp.floor((d + 0.5 * ld) * inv_ld)
        return d - off * ld

    dx = wrapped(cx, rx, ldx, inv_ldx)
    dy = wrapped(cy, ry, ldy, inv_ldy)
    dz = wrapped(cz, rz, ldz, inv_ldz)
    sq = dx * dx + dy * dy + dz * dz

    row_ids = rb * n_block + jax.lax.broadcasted_iota(
        jnp.int32, (n_block, n_atoms), 0)
    col_ids = jax.lax.broadcasted_iota(jnp.int32, (n_block, n_atoms), 1)
    mask = (col_ids > row_ids) & (sq < CUTOFF * CUTOFF) & (sq != 0.0)

    inv_sq = 1.0 / jnp.where(mask, sq, 1.0)
    sr6 = (SIGMA * SIGMA) * inv_sq
    sr6 = sr6 * sr6 * sr6
    w = (24.0 * EPSILON) * (2.0 * sr6 * sr6 - sr6) * inv_sq
    w = jnp.where(mask, w, 0.0)

    dxw = dx * w
    sxy = jnp.sum(dxw * dy).reshape(1, 1)
    sxz = jnp.sum(dxw * dz).reshape(1, 1)
    syz = jnp.sum(dy * w * dz).reshape(1, 1)
    part = jnp.concatenate([sxy, sxz, syz], axis=1) * vir_fac

    @pl.when(rb == 0)
    def _():
        mm = m[0]
        vxm = vx[0] * mm
        kxy = jnp.sum(vxm * vy[0]).reshape(1, 1)
        kxz = jnp.sum(vxm * vz[0]).reshape(1, 1)
        kyz = jnp.sum(vy[0] * mm * vz[0]).reshape(1, 1)
        kin = jnp.concatenate([kxy, kxz, kyz], axis=1) * kin_fac
        out_ref[0] = part + kin

    @pl.when(rb != 0)
    def _():
        out_ref[0] = part


def kernel(mass, y, cell):
    T = y.shape[0]
    n = y.shape[1] // 2
    V = y[:, :n]
    Q = y[:, n:]

    vol = jnp.linalg.det(cell) * 1e-30
    unit_conversion = 1.0 / 0.001987191 * 1.380649 * 1e-23
    c = 6.946704300182635e-24
    kin_fac = unit_conversion / vol
    vir_fac = 2.0 / vol * c
    ld = jnp.diagonal(cell)
    params = jnp.concatenate(
        [ld, 1.0 / ld, jnp.stack([kin_fac, vir_fac])]).astype(jnp.float32)

    qr = [Q[:, :, d, None] for d in range(3)]
    qc = [Q[:, None, :, d] for d in range(3)]
    vc = [V[:, None, :, d] for d in range(3)]
    m = mass[None, None, :, 0]

    rb = n // N_BLOCK
    grid = (T, rb)

    row_spec = pl.BlockSpec((1, N_BLOCK, 1), lambda t, r: (t, r, 0))
    col_spec = pl.BlockSpec((1, 1, n), lambda t, r: (t, 0, 0))
    bcast_spec = pl.BlockSpec((1, 1, n), lambda t, r: (0, 0, 0))

    out = pl.pallas_call(
        functools.partial(_pressure_kernel, n_atoms=n, n_block=N_BLOCK),
        grid=grid,
        in_specs=[
            pl.BlockSpec(memory_space=pltpu.SMEM),
            row_spec, row_spec, row_spec,
            col_spec, col_spec, col_spec,
            col_spec, col_spec, col_spec,
            bcast_spec,
        ],
        out_specs=pl.BlockSpec((1, 1, 3), lambda t, r: (t, r, 0)),
        out_shape=jax.ShapeDtypeStruct((T, rb, 3), jnp.float32),
    )(params, *qr, *qc, *vc, m)

    return jnp.sum(out, axis=1)


# speedup vs baseline: 1.1979x; 1.1979x over previous
import functools

import jax
import jax.numpy as jnp
from jax.experimental import pallas as pl
from jax.experimental.pallas import tpu as pltpu

CUTOFF = 9.0
SIGMA = 3.405
EPSILON = 0.238

N_BLOCK = 256


def _pressure_kernel(params_ref, qxr, qyr, qzr, qxc, qyc, qzc,
                     vx, vy, vz, m, vir_ref, kin_ref, *, n_atoms, n_block):
    rb = pl.program_id(1)
    ldx = params_ref[0]
    ldy = params_ref[1]
    ldz = params_ref[2]
    inv_ldx = params_ref[3]
    inv_ldy = params_ref[4]
    inv_ldz = params_ref[5]

    rx = qxr[0]
    ry = qyr[0]
    rz = qzr[0]
    cx = qxc[0]
    cy = qyc[0]
    cz = qzc[0]

    def wrapped(c, r, ld, inv_ld):
        d = c - r
        off = jnp.floor((d + 0.5 * ld) * inv_ld)
        return d - off * ld

    dx = wrapped(cx, rx, ldx, inv_ldx)
    dy = wrapped(cy, ry, ldy, inv_ldy)
    dz = wrapped(cz, rz, ldz, inv_ldz)
    sq = dx * dx + dy * dy + dz * dz

    row_ids = rb * n_block + jax.lax.broadcasted_iota(
        jnp.int32, (n_block, n_atoms), 0)
    col_ids = jax.lax.broadcasted_iota(jnp.int32, (n_block, n_atoms), 1)
    mask = (col_ids > row_ids) & (sq < CUTOFF * CUTOFF) & (sq != 0.0)

    inv_sq = 1.0 / jnp.where(mask, sq, 1.0)
    sr6 = (SIGMA * SIGMA) * inv_sq
    sr6 = sr6 * sr6 * sr6
    w = (24.0 * EPSILON) * (2.0 * sr6 * sr6 - sr6) * inv_sq
    w = jnp.where(mask, w, 0.0)

    dxw = dx * w
    sxy = jnp.sum(dxw * dy).reshape(1, 1)
    sxz = jnp.sum(dxw * dz).reshape(1, 1)
    syz = jnp.sum(dy * w * dz).reshape(1, 1)
    vir_ref[0, 0] = jnp.concatenate([sxy, sxz, syz], axis=1)

    @pl.when(rb == 0)
    def _():
        mm = m[0]
        vxm = vx[0] * mm
        kxy = jnp.sum(vxm * vy[0]).reshape(1, 1)
        kxz = jnp.sum(vxm * vz[0]).reshape(1, 1)
        kyz = jnp.sum(vy[0] * mm * vz[0]).reshape(1, 1)
        kin_ref[0, 0] = jnp.concatenate([kxy, kxz, kyz], axis=1)


def kernel(mass, y, cell):
    T = y.shape[0]
    n = y.shape[1] // 2
    V = y[:, :n]
    Q = y[:, n:]

    vol = jnp.linalg.det(cell) * 1e-30
    unit_conversion = 1.0 / 0.001987191 * 1.380649 * 1e-23
    c = 6.946704300182635e-24
    ld = jnp.diagonal(cell)
    params = jnp.concatenate([ld, 1.0 / ld]).astype(jnp.float32)

    qr = [Q[:, :, d, None] for d in range(3)]
    qc = [Q[:, None, :, d] for d in range(3)]
    vc = [V[:, None, :, d] for d in range(3)]
    m = mass[None, None, :, 0]

    rb = n // N_BLOCK
    grid = (T, rb)

    row_spec = pl.BlockSpec((1, N_BLOCK, 1), lambda t, r: (t, r, 0))
    col_spec = pl.BlockSpec((1, 1, n), lambda t, r: (t, 0, 0))
    bcast_spec = pl.BlockSpec((1, 1, n), lambda t, r: (0, 0, 0))

    vir, kin = pl.pallas_call(
        functools.partial(_pressure_kernel, n_atoms=n, n_block=N_BLOCK),
        grid=grid,
        in_specs=[
            pl.BlockSpec(memory_space=pltpu.SMEM),
            row_spec, row_spec, row_spec,
            col_spec, col_spec, col_spec,
            col_spec, col_spec, col_spec,
            bcast_spec,
        ],
        out_specs=[
            pl.BlockSpec((1, 1, 1, 3), lambda t, r: (t, r, 0, 0)),
            pl.BlockSpec((1, 1, 1, 3), lambda t, r: (t, 0, 0, 0)),
        ],
        out_shape=[
            jax.ShapeDtypeStruct((T, rb, 1, 3), jnp.float32),
            jax.ShapeDtypeStruct((T, 1, 1, 3), jnp.float32),
        ],
    )(params, *qr, *qc, *vc, m)

    p = kin[:, 0, 0] / vol * unit_conversion
    v = jnp.sum(vir[:, :, 0], axis=1) * 2.0 / vol * c
    return p + v
